# baseline (speedup 1.0000x reference)
import functools

import jax
import jax.numpy as jnp
from jax import lax
from jax.experimental import pallas as pl
from jax.experimental.pallas import tpu as pltpu
from jax.experimental.pallas import tpu_sc as plsc

HASH_BUCKETS = 250000
EMBED_DIM = 64
CHUNK = 128
K = 2
GROUP = K * CHUNK
S = 5
LANES = 16
NW = 32


def _emb_body(n_groups, x_hbm, table_hbm, out_hbm, *scratch):
    idx = scratch[0:S]
    rows = scratch[S:2 * S]
    gsem = scratch[2 * S:3 * S]
    wsem = scratch[3 * S:4 * S]
    wid = lax.axis_index("s") * 2 + lax.axis_index("c")
    base_row = wid * (n_groups * GROUP)

    def load_idx(s, g):
        row0 = base_row + g * GROUP
        pltpu.sync_copy(x_hbm.at[pl.ds(row0, GROUP)], idx[s])

        def mod_one(j, carry):
            sl = pl.ds(j * LANES, LANES)
            idx[s][sl] = lax.rem(idx[s][sl], HASH_BUCKETS)
            return carry

        lax.fori_loop(0, GROUP // LANES, mod_one, 0)

    def fire_gathers(s):
        for b in range(K):
            pltpu.async_copy(table_hbm.at[idx[s].at[pl.ds(b * CHUNK, CHUNK)]],
                             rows[s].at[pl.ds(b * CHUNK, CHUNK)], gsem[s])

    def drain_gathers(s):
        pltpu.make_async_copy(out_hbm.at[pl.ds(base_row, GROUP)], rows[s],
                              gsem[s]).wait()

    def fire_write(s, g):
        row0 = base_row + g * GROUP
        pltpu.async_copy(rows[s], out_hbm.at[pl.ds(row0, GROUP)], wsem[s])

    def wait_write(s):
        pltpu.make_async_copy(out_hbm.at[pl.ds(base_row, GROUP)], rows[s],
                              wsem[s]).wait()

    def refill(s, g):
        wait_write(s)
        load_idx(s, g)
        fire_gathers(s)

    for s in range(S):
        load_idx(s, s)
        fire_gathers(s)

    for s in range(S):
        drain_gathers(s)
        fire_write(s, s)
        if s >= 1:
            refill(s - 1, s - 1 + S)

    n_blocks = n_groups // S

    def block(i, carry):
        for s in range(S):
            g = i * S + s
            drain_gathers(s)
            fire_write(s, g)
            refill((s - 1) % S, g + S - 1)
        return carry

    lax.fori_loop(1, n_blocks - 1, block, 0)

    for s in range(S):
        g = (n_blocks - 1) * S + s
        drain_gathers(s)
        fire_write(s, g)
        if s == 0:
            refill(S - 1, g + S - 1)

    for s in range(S):
        wait_write(s)


def kernel(x, table):
    rows, cols = x.shape
    b = rows * cols
    xf = x.reshape(b).astype(jnp.int32)
    assert b % (NW * GROUP) == 0
    n_groups = b // (NW * GROUP)
    assert n_groups % S == 0 and n_groups // S >= 2

    mesh = plsc.VectorSubcoreMesh(core_axis_name="c", subcore_axis_name="s")
    run = functools.partial(
        pl.kernel,
        mesh=mesh,
        compiler_params=pltpu.CompilerParams(use_tc_tiling_on_sc=False),
        out_type=jax.ShapeDtypeStruct((b, EMBED_DIM), jnp.float32),
        scratch_types=(
            [pltpu.VMEM((GROUP,), jnp.int32) for _ in range(S)]
            + [pltpu.VMEM((GROUP, EMBED_DIM), jnp.float32) for _ in range(S)]
            + [pltpu.SemaphoreType.DMA for _ in range(2 * S)]
        ),
    )(functools.partial(_emb_body, n_groups))
    out = run(xf, table)
    return out.reshape(rows, cols, EMBED_DIM)
